# Initial kernel scaffold; baseline (speedup 1.0000x reference)
import jax
import jax.numpy as jnp
from jax.experimental import pallas as pl


def kernel(weekend, table):
    raise NotImplementedError("write your pallas kernel here")


# speedup vs baseline: 3.2024x; 3.2024x over previous
import jax
import jax.numpy as jnp
from jax import lax
from jax.experimental import pallas as pl
from jax.experimental.pallas import tpu as pltpu, tpu_sc as plsc

EMBED = 64
NC, NS = 2, 16
NW = NC * NS
_Q = 4
_C = 128
_QC = _Q * _C


def _sc_body(t4_hbm, idx_hbm, out_hbm,
             idx_raw, cidx0, cidx1, rows0, rows1, sg0, sg1, sw0, sw1):
    wid = lax.axis_index("s") * NC + lax.axis_index("c")
    total_c = out_hbm.shape[0]
    per_w = total_c // NW
    nb2 = per_w // (2 * _C)
    c0 = wid * per_w

    def compute_cidx(raw_off, cref):
        for g in range(_C // 16):
            o = raw_off + g * 16
            v0 = idx_raw[0, pl.ds(o, 16)]
            v1 = idx_raw[1, pl.ds(o, 16)]
            v2 = idx_raw[2, pl.ds(o, 16)]
            v3 = idx_raw[3, pl.ds(o, 16)]
            cref[pl.ds(g * 16, 16)] = ((v0 * 3 + v1) * 3 + v2) * 3 + v3

    def body(i, carry):
        ca = c0 + i * 2 * _C
        cb = ca + _C
        for j in range(_Q):
            pltpu.sync_copy(idx_hbm.at[j].at[pl.ds(ca, 2 * _C)],
                            idx_raw.at[j])
        compute_cidx(0, cidx0)

        @pl.when(i > 0)
        def _():
            pltpu.make_async_copy(rows0, out_hbm.at[pl.ds(0, _C)], sw0).wait()

        ga = pltpu.async_copy(t4_hbm.at[cidx0], rows0, sg0)
        compute_cidx(_C, cidx1)

        @pl.when(i > 0)
        def _():
            pltpu.make_async_copy(rows1, out_hbm.at[pl.ds(0, _C)], sw1).wait()

        gb = pltpu.async_copy(t4_hbm.at[cidx1], rows1, sg1)
        ga.wait()
        pltpu.make_async_copy(rows0, out_hbm.at[pl.ds(ca, _C)], sw0).start()
        gb.wait()
        pltpu.make_async_copy(rows1, out_hbm.at[pl.ds(cb, _C)], sw1).start()
        return carry

    lax.fori_loop(0, nb2, body, 0)
    pltpu.make_async_copy(rows0, out_hbm.at[pl.ds(0, _C)], sw0).wait()
    pltpu.make_async_copy(rows1, out_hbm.at[pl.ds(0, _C)], sw1).wait()


def kernel(weekend, table):
    n, m = weekend.shape
    b = n * m
    bc = b // _Q
    idx_t = weekend.reshape(bc, _Q).T.astype(jnp.int32)
    code = jnp.arange(3 ** _Q, dtype=jnp.int32)
    t4 = jnp.concatenate(
        [table[(code // (3 ** (_Q - 1 - j))) % 3] for j in range(_Q)], axis=1)
    k = pl.kernel(
        _sc_body,
        out_type=jax.ShapeDtypeStruct((bc, _Q * EMBED), jnp.float32),
        mesh=plsc.VectorSubcoreMesh(core_axis_name="c", subcore_axis_name="s"),
        scratch_types=[
            pltpu.VMEM((_Q, 2 * _C), jnp.int32),
            pltpu.VMEM((_C,), jnp.int32),
            pltpu.VMEM((_C,), jnp.int32),
            pltpu.VMEM((_C, _Q * EMBED), jnp.float32),
            pltpu.VMEM((_C, _Q * EMBED), jnp.float32),
            pltpu.SemaphoreType.DMA,
            pltpu.SemaphoreType.DMA,
            pltpu.SemaphoreType.DMA,
            pltpu.SemaphoreType.DMA,
        ],
    )
    out = k(t4, idx_t)
    return out.reshape(n, m, EMBED)


# speedup vs baseline: 23.0542x; 7.1990x over previous
import jax
import jax.numpy as jnp
from jax import lax
from jax.experimental import pallas as pl
from jax.experimental.pallas import tpu as pltpu, tpu_sc as plsc

EMBED = 64
NC, NS = 2, 16
NW = NC * NS
_S = 512
_VB = 4


def _sc_body(tbl_hbm, wt_hbm, out_hbm,
             tbl, idx0, idx1, buf0, buf1, si0, si1, so0, so1):
    wid = lax.axis_index("s") * NC + lax.axis_index("c")
    i0 = wid * _S
    nj = wt_hbm.shape[0]

    pltpu.sync_copy(tbl_hbm, tbl)
    pltpu.async_copy(wt_hbm.at[0, pl.ds(i0, _S)], idx0, si0)
    pltpu.async_copy(wt_hbm.at[1, pl.ds(i0, _S)], idx1, si1)

    def compute(idx_v, buf):
        def vb_body(vb, carry):
            base = vb * 128
            m0 = []
            m1 = []
            for k in range(8):
                iv = idx_v[pl.ds(base + 16 * k, 16)]
                m0.append(iv == 0)
                m1.append(iv == 1)
            for e in range(EMBED):
                t0 = tbl[0, e, :]
                t1 = tbl[1, e, :]
                t2 = tbl[2, e, :]
                for k in range(8):
                    buf[e, pl.ds(base + 16 * k, 16)] = jnp.where(
                        m0[k], t0, jnp.where(m1[k], t1, t2))
            return carry
        lax.fori_loop(0, _VB, vb_body, 0)

    def half(j, idx_v, buf, si, so):
        pltpu.make_async_copy(wt_hbm.at[0, pl.ds(0, _S)], idx_v, si).wait()

        @pl.when(j >= 2)
        def _():
            pltpu.make_async_copy(
                buf, out_hbm.at[0, :, pl.ds(0, _S)], so).wait()

        compute(idx_v, buf)
        pltpu.make_async_copy(
            buf, out_hbm.at[j, :, pl.ds(i0, _S)], so).start()

        @pl.when(j + 2 < nj)
        def _():
            pltpu.async_copy(wt_hbm.at[j + 2, pl.ds(i0, _S)], idx_v, si)

    def body(t, carry):
        half(2 * t, idx0, buf0, si0, so0)
        half(2 * t + 1, idx1, buf1, si1, so1)
        return carry

    lax.fori_loop(0, nj // 2, body, 0)
    pltpu.make_async_copy(buf0, out_hbm.at[0, :, pl.ds(0, _S)], so0).wait()
    pltpu.make_async_copy(buf1, out_hbm.at[0, :, pl.ds(0, _S)], so1).wait()


def kernel(weekend, table):
    n, m = weekend.shape
    wt = weekend.T.astype(jnp.int32)
    tbl = jnp.broadcast_to(table[:, :, None], (3, EMBED, 16))
    k = pl.kernel(
        _sc_body,
        out_type=jax.ShapeDtypeStruct((m, EMBED, n), jnp.float32),
        mesh=plsc.VectorSubcoreMesh(core_axis_name="c", subcore_axis_name="s"),
        scratch_types=[
            pltpu.VMEM((3, EMBED, 16), jnp.float32),
            pltpu.VMEM((_S,), jnp.int32),
            pltpu.VMEM((_S,), jnp.int32),
            pltpu.VMEM((EMBED, _S), jnp.float32),
            pltpu.VMEM((EMBED, _S), jnp.float32),
            pltpu.SemaphoreType.DMA,
            pltpu.SemaphoreType.DMA,
            pltpu.SemaphoreType.DMA,
            pltpu.SemaphoreType.DMA,
        ],
    )
    out = k(tbl, wt)
    return out.transpose(2, 0, 1)


# speedup vs baseline: 23.2078x; 1.0067x over previous
import jax
import jax.numpy as jnp
from jax import lax
from jax.experimental import pallas as pl
from jax.experimental.pallas import tpu as pltpu, tpu_sc as plsc

EMBED = 64
NC, NS = 2, 16
NW = NC * NS
_S = 512
_VB = 4


def _sc_body(tbl_hbm, wt_hbm, out_hbm,
             tbl, idx0, idx1, buf0, buf1, si0, si1, so0, so1):
    wid = lax.axis_index("s") * NC + lax.axis_index("c")
    i0 = wid * _S
    nj = wt_hbm.shape[0]

    pltpu.sync_copy(tbl_hbm, tbl)
    pltpu.async_copy(wt_hbm.at[0, pl.ds(i0, _S)], idx0, si0)
    pltpu.async_copy(wt_hbm.at[1, pl.ds(i0, _S)], idx1, si1)

    def compute(idx_v, buf, e0):
        def vb_body(vb, carry):
            base = vb * 128
            m0 = []
            m1 = []
            for k in range(8):
                iv = idx_v[pl.ds(base + 16 * k, 16)]
                m0.append(iv == 0)
                m1.append(iv == 1)
            for e in range(e0, e0 + EMBED // 2):
                t0 = tbl[0, e, :]
                t1 = tbl[1, e, :]
                t2 = tbl[2, e, :]
                for k in range(8):
                    buf[e, pl.ds(base + 16 * k, 16)] = jnp.where(
                        m0[k], t0, jnp.where(m1[k], t1, t2))
            return carry
        lax.fori_loop(0, _VB, vb_body, 0)

    def half(j, idx_v, buf, si, so):
        pltpu.make_async_copy(wt_hbm.at[0, pl.ds(0, _S)], idx_v, si).wait()

        @pl.when(j >= 2)
        def _():
            for _h in range(2):
                pltpu.make_async_copy(
                    buf.at[pl.ds(0, EMBED // 2)],
                    out_hbm.at[0, pl.ds(0, EMBED // 2), pl.ds(0, _S)],
                    so).wait()

        compute(idx_v, buf, 0)
        pltpu.make_async_copy(
            buf.at[pl.ds(0, EMBED // 2)],
            out_hbm.at[j, pl.ds(0, EMBED // 2), pl.ds(i0, _S)], so).start()
        compute(idx_v, buf, EMBED // 2)
        pltpu.make_async_copy(
            buf.at[pl.ds(EMBED // 2, EMBED // 2)],
            out_hbm.at[j, pl.ds(EMBED // 2, EMBED // 2), pl.ds(i0, _S)],
            so).start()

        @pl.when(j + 2 < nj)
        def _():
            pltpu.async_copy(wt_hbm.at[j + 2, pl.ds(i0, _S)], idx_v, si)

    def body(t, carry):
        half(2 * t, idx0, buf0, si0, so0)
        half(2 * t + 1, idx1, buf1, si1, so1)
        return carry

    lax.fori_loop(0, nj // 2, body, 0)
    for so in (so0, so1):
        for _h in range(2):
            pltpu.make_async_copy(
                buf0.at[pl.ds(0, EMBED // 2)],
                out_hbm.at[0, pl.ds(0, EMBED // 2), pl.ds(0, _S)], so).wait()


def kernel(weekend, table):
    n, m = weekend.shape
    wt = weekend.T.astype(jnp.int32)
    tbl = jnp.broadcast_to(table[:, :, None], (3, EMBED, 16))
    k = pl.kernel(
        _sc_body,
        out_type=jax.ShapeDtypeStruct((m, EMBED, n), jnp.float32),
        mesh=plsc.VectorSubcoreMesh(core_axis_name="c", subcore_axis_name="s"),
        scratch_types=[
            pltpu.VMEM((3, EMBED, 16), jnp.float32),
            pltpu.VMEM((_S,), jnp.int32),
            pltpu.VMEM((_S,), jnp.int32),
            pltpu.VMEM((EMBED, _S), jnp.float32),
            pltpu.VMEM((EMBED, _S), jnp.float32),
            pltpu.SemaphoreType.DMA,
            pltpu.SemaphoreType.DMA,
            pltpu.SemaphoreType.DMA,
            pltpu.SemaphoreType.DMA,
        ],
    )
    out = k(tbl, wt)
    return out.transpose(2, 0, 1)
